# baseline (speedup 1.0000x reference)
import math

import jax
import jax.numpy as jnp
from jax import lax
from jax.experimental import pallas as pl
from jax.experimental.pallas import tpu as pltpu, tpu_sc as plsc

B, T = 4, 8192
D_MODEL = 1024
D_HALF = 512
POS_SIZE = 128
PITCH_START = 128
PITCH_SIZE = 32

NUM_CORES = 2
NUM_SUBCORES = 16
NUM_TILES = NUM_CORES * NUM_SUBCORES
BT = B * T
TOK_PER_TILE = BT // NUM_TILES
CHUNKS_PER_ROW = T // TOK_PER_TILE
VECS_PER_TILE = TOK_PER_TILE // 16
GATHER_K = 32
BURSTS = TOK_PER_TILE // GATHER_K

ZERO_L = POS_SIZE * 2
ZERO_R = PITCH_SIZE


_GATHER_DNUMS = lax.GatherDimensionNumbers(
    offset_dims=(), collapsed_slice_dims=(0,), start_index_map=(0,))


def _take(v, idx):
    return lax.gather(v, idx[:, None], _GATHER_DNUMS, slice_sizes=(1,),
                      mode=lax.GatherScatterMode.PROMISE_IN_BOUNDS)


def _body(ids_hbm, ltab_hbm, rtab_hbm, out_hbm,
          ids_v, lidx_v, ridx_v, buf0, buf1, sem_l, sem_r):
    cid = lax.axis_index("c")
    sid = lax.axis_index("s")
    wid = sid * NUM_CORES + cid
    row = wid // CHUNKS_PER_ROW
    chunk = wid % CHUNKS_PER_ROW

    pltpu.sync_copy(ids_hbm.at[pl.ds(row * T, T)], ids_v)

    iota16 = lax.iota(jnp.int32, 16)
    lane15 = jnp.full((16,), 15, jnp.int32)

    def lane_cummax(v):
        for d in (1, 2, 4, 8):
            v = jnp.maximum(v, _take(v, jnp.maximum(iota16 - d, 0)))
        return v

    def pre_body(i, m):
        ids16 = ids_v[pl.ds(i * 16, 16)]
        t16 = i * 16 + iota16
        enc = jnp.where(ids16 < POS_SIZE, (t16 << 7) | ids16, -1)
        return jnp.maximum(m, enc)

    pre = lax.fori_loop(0, chunk * VECS_PER_TILE, pre_body,
                        jnp.full((16,), -1, jnp.int32))
    carry0 = _take(lane_cummax(pre), lane15)

    base = chunk * TOK_PER_TILE

    def scan_body(j, carry):
        off = base + j * 16
        ids16 = ids_v[pl.ds(off, 16)]
        t16 = off + iota16
        pos = ids16 < POS_SIZE
        pitch = jnp.logical_and(ids16 >= PITCH_START,
                                ids16 < PITCH_START + PITCH_SIZE)
        enc = jnp.where(pos, (t16 << 7) | ids16, -1)
        cm = jnp.maximum(lane_cummax(enc), carry)
        ff = jnp.maximum(cm, 0) & (POS_SIZE - 1)
        li = jnp.where(pos, ids16,
                       jnp.where(pitch, ff + POS_SIZE, ZERO_L))
        ri = jnp.where(pitch, ids16 - PITCH_START, ZERO_R)
        lidx_v[pl.ds(j * 16, 16)] = li
        ridx_v[pl.ds(j * 16, 16)] = ri
        return _take(cm, lane15)

    lax.fori_loop(0, VECS_PER_TILE, scan_body, carry0)

    out_base = wid * TOK_PER_TILE

    def issue(k, buf, sem):
        idx = pl.ds(k * GATHER_K, GATHER_K)
        pltpu.async_copy(ltab_hbm.at[lidx_v.at[idx]],
                         buf.at[:, pl.ds(0, D_HALF)], sem)
        pltpu.async_copy(rtab_hbm.at[ridx_v.at[idx]],
                         buf.at[:, pl.ds(D_HALF, D_HALF)], sem)

    def drain(k, buf, sem):
        idx = pl.ds(k * GATHER_K, GATHER_K)
        pltpu.make_async_copy(ltab_hbm.at[lidx_v.at[idx]],
                              buf.at[:, pl.ds(0, D_HALF)], sem).wait()
        pltpu.make_async_copy(rtab_hbm.at[ridx_v.at[idx]],
                              buf.at[:, pl.ds(D_HALF, D_HALF)], sem).wait()

    def write(k, buf):
        pltpu.sync_copy(buf, out_hbm.at[pl.ds(out_base + k * GATHER_K,
                                              GATHER_K), :])

    issue(0, buf0, sem_l)

    def g_body(h, _):
        k0 = 2 * h
        k1 = k0 + 1
        issue(k1, buf1, sem_r)
        drain(k0, buf0, sem_l)
        write(k0, buf0)

        @pl.when(k1 + 1 < BURSTS)
        def _():
            issue(k1 + 1, buf0, sem_l)

        drain(k1, buf1, sem_r)
        write(k1, buf1)
        return 0

    lax.fori_loop(0, BURSTS // 2, g_body, 0)


_sc_kernel = pl.kernel(
    _body,
    out_type=jax.ShapeDtypeStruct((BT, D_MODEL), jnp.float32),
    mesh=plsc.VectorSubcoreMesh(core_axis_name="c", subcore_axis_name="s"),
    scratch_types=[
        pltpu.VMEM((T,), jnp.int32),
        pltpu.VMEM((TOK_PER_TILE,), jnp.int32),
        pltpu.VMEM((TOK_PER_TILE,), jnp.int32),
        pltpu.VMEM((GATHER_K, D_MODEL), jnp.float32),
        pltpu.VMEM((GATHER_K, D_MODEL), jnp.float32),
        pltpu.SemaphoreType.DMA,
        pltpu.SemaphoreType.DMA,
    ],
)


@jax.jit
def kernel(token_ids, table_pos, table_pitch):
    sqrt2 = jnp.float32(math.sqrt(2.0))
    ltab = jnp.concatenate(
        [table_pos * sqrt2, table_pos,
         jnp.zeros((1, D_HALF), jnp.float32)], axis=0)
    rtab = jnp.concatenate(
        [table_pitch, jnp.zeros((1, D_HALF), jnp.float32)], axis=0)
    out = _sc_kernel(token_ids.reshape(BT), ltab, rtab)
    return out.reshape(B, T, D_MODEL)


# speedup vs baseline: 3.9220x; 3.9220x over previous
import math

import jax
import jax.numpy as jnp
from jax import lax
from jax.experimental import pallas as pl
from jax.experimental.pallas import tpu as pltpu, tpu_sc as plsc

B, T = 4, 8192
D_MODEL = 1024
D_HALF = 512
POS_SIZE = 128
PITCH_START = 128
PITCH_SIZE = 32

NUM_CORES = 2
NUM_SUBCORES = 16
NUM_TILES = NUM_CORES * NUM_SUBCORES
BT = B * T
TOK_PER_TILE = BT // NUM_TILES
CHUNKS_PER_ROW = T // TOK_PER_TILE
VECS_PER_TILE = TOK_PER_TILE // 16
GROUPS_PER_TILE = VECS_PER_TILE

TAB_WORDS = (POS_SIZE + PITCH_SIZE) * D_HALF
TPI_OFF = POS_SIZE * D_HALF
ZERO_L = POS_SIZE * 2
ZERO_R = PITCH_SIZE
BUF_WORDS = 16 * D_MODEL

_GATHER_DNUMS = lax.GatherDimensionNumbers(
    offset_dims=(), collapsed_slice_dims=(0,), start_index_map=(0,))


def _take(v, idx):
    return lax.gather(v, idx[:, None], _GATHER_DNUMS, slice_sizes=(1,),
                      mode=lax.GatherScatterMode.PROMISE_IN_BOUNDS)


def _body(ids_hbm, tpos_hbm, tpit_hbm, out_hbm,
          ids_v, tab_v, buf0, buf1, idx_s, wsem0, wsem1):
    cid = lax.axis_index("c")
    sid = lax.axis_index("s")
    wid = sid * NUM_CORES + cid
    row = wid // CHUNKS_PER_ROW
    chunk = wid % CHUNKS_PER_ROW

    pltpu.sync_copy(ids_hbm.at[pl.ds(row * T, T)], ids_v)
    pltpu.sync_copy(tpos_hbm, tab_v.at[pl.ds(0, TPI_OFF)])
    pltpu.sync_copy(tpit_hbm, tab_v.at[pl.ds(TPI_OFF, PITCH_SIZE * D_HALF)])

    iota16 = lax.iota(jnp.int32, 16)
    lane15 = jnp.full((16,), 15, jnp.int32)

    def lane_cummax(v):
        for d in (1, 2, 4, 8):
            v = jnp.maximum(v, _take(v, jnp.maximum(iota16 - d, 0)))
        return v

    def pre_body(i, m):
        ids16 = ids_v[pl.ds(i * 16, 16)]
        t16 = i * 16 + iota16
        enc = jnp.where(ids16 < POS_SIZE, (t16 << 7) | ids16, -1)
        return jnp.maximum(m, enc)

    pre = lax.fori_loop(0, chunk * VECS_PER_TILE, pre_body,
                        jnp.full((16,), -1, jnp.int32))
    carry0 = _take(lane_cummax(pre), lane15)

    base = chunk * TOK_PER_TILE

    def scan_body(j, carry):
        off = base + j * 16
        ids16 = ids_v[pl.ds(off, 16)]
        t16 = off + iota16
        pos = ids16 < POS_SIZE
        pitch = jnp.logical_and(ids16 >= PITCH_START,
                                ids16 < PITCH_START + PITCH_SIZE)
        enc = jnp.where(pos, (t16 << 7) | ids16, -1)
        cm = jnp.maximum(lane_cummax(enc), carry)
        ff = jnp.maximum(cm, 0) & (POS_SIZE - 1)
        li = jnp.where(pos, ids16,
                       jnp.where(pitch, ff + POS_SIZE, ZERO_L))
        ri = jnp.where(pitch, ids16 - PITCH_START, ZERO_R)
        packed = li | (ri << 9)
        for lane in range(16):
            idx_s[j * 16 + lane] = packed[lane]
        return _take(cm, lane15)

    lax.fori_loop(0, GROUPS_PER_TILE, scan_body, carry0)

    sqrt2 = jnp.float32(math.sqrt(2.0))
    one = jnp.float32(1.0)
    zv = jnp.zeros((16,), jnp.float32)
    out_base = wid * TOK_PER_TILE * D_MODEL

    def fill_group(g, buf):
        def tok(t, _):
            p = idx_s[g * 16 + t]
            l = p & 0x1FF
            r = p >> 9
            tbase = t << 10

            @pl.when(l < ZERO_L)
            def _():
                sc = jnp.broadcast_to(
                    lax.select(l < POS_SIZE, sqrt2, one), (16,))
                la = (l & (POS_SIZE - 1)) << 9
                for c in range(32):
                    buf[pl.ds(tbase + c * 16, 16)] = (
                        tab_v[pl.ds(la + c * 16, 16)] * sc)

            @pl.when(l >= ZERO_L)
            def _():
                for c in range(32):
                    buf[pl.ds(tbase + c * 16, 16)] = zv

            @pl.when(r < ZERO_R)
            def _():
                ra = TPI_OFF + (r << 9)
                for c in range(32):
                    buf[pl.ds(tbase + D_HALF + c * 16, 16)] = (
                        tab_v[pl.ds(ra + c * 16, 16)])

            @pl.when(r >= ZERO_R)
            def _():
                for c in range(32):
                    buf[pl.ds(tbase + D_HALF + c * 16, 16)] = zv

            return 0

        lax.fori_loop(0, 16, tok, 0)

    def out_slice(g):
        return out_hbm.at[pl.ds(out_base + g * BUF_WORDS, BUF_WORDS)]

    def pair_body(h, _):
        g0 = 2 * h
        g1 = g0 + 1

        @pl.when(h > 0)
        def _():
            pltpu.make_async_copy(buf0, out_slice(g0), wsem0).wait()

        fill_group(g0, buf0)
        pltpu.async_copy(buf0, out_slice(g0), wsem0)

        @pl.when(h > 0)
        def _():
            pltpu.make_async_copy(buf1, out_slice(g1), wsem1).wait()

        fill_group(g1, buf1)
        pltpu.async_copy(buf1, out_slice(g1), wsem1)
        return 0

    lax.fori_loop(0, GROUPS_PER_TILE // 2, pair_body, 0)
    pltpu.make_async_copy(buf0, out_slice(0), wsem0).wait()
    pltpu.make_async_copy(buf1, out_slice(1), wsem1).wait()


_sc_kernel = pl.kernel(
    _body,
    out_type=jax.ShapeDtypeStruct((BT * D_MODEL,), jnp.float32),
    mesh=plsc.VectorSubcoreMesh(core_axis_name="c", subcore_axis_name="s"),
    scratch_types=[
        pltpu.VMEM((T,), jnp.int32),
        pltpu.VMEM((TAB_WORDS,), jnp.float32),
        pltpu.VMEM((BUF_WORDS,), jnp.float32),
        pltpu.VMEM((BUF_WORDS,), jnp.float32),
        pltpu.SMEM((TOK_PER_TILE,), jnp.int32),
        pltpu.SemaphoreType.DMA,
        pltpu.SemaphoreType.DMA,
    ],
)


@jax.jit
def kernel(token_ids, table_pos, table_pitch):
    out = _sc_kernel(token_ids.reshape(BT), table_pos.reshape(-1),
                     table_pitch.reshape(-1))
    return out.reshape(B, T, D_MODEL)
